# baseline (speedup 1.0000x reference)
import dataclasses
import functools

import jax
import numpy as np
import jax.numpy as jnp
from jax import lax
from jax.experimental import pallas as pl
from jax.experimental.pallas import tpu as pltpu
from jax.experimental.pallas import tpu_sc as plsc

_NCORES = 2
_NW = 16 * _NCORES
_LANES = 16


def _sc_gather(pos_table, idx_flat):
    n = idx_flat.shape[0]
    v, h = pos_table.shape
    n_per = n // _NW
    groups = n_per // _LANES
    idx2d = idx_flat.reshape(_NW, n_per)
    table_flat = pos_table.reshape(1, v * h)
    mesh = plsc.VectorSubcoreMesh(core_axis_name="core",
                                  subcore_axis_name="subcore",
                                  num_cores=_NCORES)

    cp = pltpu.CompilerParams()
    if "needs_layout_passes" in pltpu.CompilerParams.__dataclass_fields__:
        cp = dataclasses.replace(cp, needs_layout_passes=False)

    @functools.partial(
        pl.kernel,
        out_type=jax.ShapeDtypeStruct((_NW, n_per * h), jnp.float32),
        mesh=mesh,
        compiler_params=cp,
        scratch_types=[
            pltpu.VMEM((v * h,), jnp.float32),
            pltpu.VMEM((n_per,), jnp.int32),
            pltpu.VMEM((n_per * h,), jnp.float32),
            pltpu.SemaphoreType.DMA,
            pltpu.SemaphoreType.DMA,
        ],
    )
    def gather_kernel(table_hbm, idx_hbm, o_hbm, table_v, idx_v, out_v,
                      sem_t, sem_i):
        cid = lax.axis_index("core")
        sid = lax.axis_index("subcore")
        wid = sid * _NCORES + cid
        ct = pltpu.async_copy(table_hbm.at[0], table_v, sem_t)
        ci = pltpu.async_copy(idx_hbm.at[wid], idx_v, sem_i)
        ci.wait()
        ct.wait()
        iota_h = lax.iota(jnp.int32, _LANES) * h

        @pl.loop(0, groups)
        def _(m):
            pos = idx_v[pl.ds(m * _LANES, _LANES)] * h
            obase = m * (_LANES * h) + iota_h
            vals = [plsc.load_gather(table_v, [pos + hh])
                    for hh in range(h)]
            for hh in range(h):
                plsc.store_scatter(out_v, [obase + hh], vals[hh])

        co = pltpu.async_copy(out_v, o_hbm.at[wid], sem_i)
        co.wait()

    return gather_kernel(table_flat, idx2d)


_BM = 128
_EXP_BLOCK = 640


def _mm_body(n_exp, window, tok_a, tok_b, w_ref, b_ref, e_ref, out_ref,
             wexp_ref):
    @pl.when(pl.program_id(0) == 0)
    def _():
        kb = _EXP_BLOCK // window
        for j in range(n_exp):
            w_slice = w_ref[pl.ds(j * kb, kb), :].astype(jnp.bfloat16)
            wexp_ref[pl.ds(j * _EXP_BLOCK, _EXP_BLOCK), :] = jnp.dot(
                e_ref[...], w_slice, preferred_element_type=jnp.float32
            ).astype(jnp.bfloat16)

    for s, tok in enumerate((tok_a, tok_b)):
        acc = jnp.dot(
            tok[...].astype(jnp.bfloat16),
            wexp_ref[...],
            preferred_element_type=jnp.float32,
        )
        out_ref[s] = acc + b_ref[...]


def _tc_matmul(tokens, w_reduce, bias_row, window):
    bm, tok_len = tokens.shape
    kw, n_out = w_reduce.shape
    n_exp = tok_len // _EXP_BLOCK
    half = bm // 2
    ng = half // _BM
    e_mat = jnp.asarray(
        np.kron(np.eye(_EXP_BLOCK // window, dtype=np.float32),
                np.full((window, 1), 1.0 / window, dtype=np.float32)),
        dtype=jnp.bfloat16)
    out = pl.pallas_call(
        functools.partial(_mm_body, n_exp, window),
        grid=(ng,),
        in_specs=[
            pl.BlockSpec((_BM, tok_len), lambda i: (i, 0)),
            pl.BlockSpec((_BM, tok_len), lambda i, _n=ng: (i + _n, 0)),
            pl.BlockSpec((kw, n_out), lambda i: (0, 0)),
            pl.BlockSpec((1, n_out), lambda i: (0, 0)),
            pl.BlockSpec(e_mat.shape, lambda i: (0, 0)),
        ],
        out_specs=pl.BlockSpec((2, _BM, n_out), lambda i: (0, i, 0)),
        out_shape=jax.ShapeDtypeStruct((2, half, n_out), jnp.float32),
        scratch_shapes=[pltpu.VMEM((tok_len, n_out), jnp.bfloat16)],
        compiler_params=pltpu.CompilerParams(
            dimension_semantics=("arbitrary",),
        ),
    )(tokens, tokens, w_reduce, bias_row, e_mat)
    return out.reshape(bm, n_out)


def _add_body(a_ref, b_ref, out_ref):
    out_ref[...] = a_ref[...] + b_ref[...]


def _tc_add(a, b):
    return pl.pallas_call(
        _add_body,
        out_shape=jax.ShapeDtypeStruct(a.shape, jnp.float32),
    )(a, b)


def kernel(semantic_tokens, semantic_pos, pos_table, W_reduce, b_reduce):
    b, tok_len = semantic_tokens.shape
    l = semantic_pos.shape[1]
    h = pos_table.shape[1]
    window = tok_len // W_reduce.shape[0]

    idx = semantic_pos.reshape(-1).astype(jnp.int32)
    pos_emb = _sc_gather(pos_table, idx).reshape(b, l * h)
    mm = _tc_matmul(semantic_tokens, W_reduce, b_reduce.reshape(1, -1),
                    window)
    return _tc_add(mm, pos_emb)


# speedup vs baseline: 1.0029x; 1.0029x over previous
import dataclasses
import functools

import jax
import numpy as np
import jax.numpy as jnp
from jax import lax
from jax.experimental import pallas as pl
from jax.experimental.pallas import tpu as pltpu
from jax.experimental.pallas import tpu_sc as plsc

_NCORES = 2
_NW = 16 * _NCORES
_LANES = 16


def _sc_gather(pos_table, idx_flat):
    n = idx_flat.shape[0]
    v, h = pos_table.shape
    n_per = n // _NW
    groups = n_per // _LANES
    idx2d = idx_flat.reshape(_NW, n_per)
    table_flat = pos_table.reshape(1, v * h)
    mesh = plsc.VectorSubcoreMesh(core_axis_name="core",
                                  subcore_axis_name="subcore",
                                  num_cores=_NCORES)

    cp = pltpu.CompilerParams()
    if "needs_layout_passes" in pltpu.CompilerParams.__dataclass_fields__:
        cp = dataclasses.replace(cp, needs_layout_passes=False)

    @functools.partial(
        pl.kernel,
        out_type=jax.ShapeDtypeStruct((_NW, n_per * h), jnp.float32),
        mesh=mesh,
        compiler_params=cp,
        scratch_types=[
            pltpu.VMEM((v * h,), jnp.float32),
            pltpu.VMEM((n_per,), jnp.int32),
            pltpu.VMEM((n_per * h,), jnp.float32),
            pltpu.SemaphoreType.DMA,
            pltpu.SemaphoreType.DMA,
        ],
    )
    def gather_kernel(table_hbm, idx_hbm, o_hbm, table_v, idx_v, out_v,
                      sem_t, sem_i):
        cid = lax.axis_index("core")
        sid = lax.axis_index("subcore")
        wid = sid * _NCORES + cid
        ct = pltpu.async_copy(table_hbm.at[0], table_v, sem_t)
        ci = pltpu.async_copy(idx_hbm.at[wid], idx_v, sem_i)
        ci.wait()
        ct.wait()
        iota_h = lax.iota(jnp.int32, _LANES) * h

        @pl.loop(0, groups)
        def _(m):
            pos = idx_v[pl.ds(m * _LANES, _LANES)] * h
            obase = m * (_LANES * h) + iota_h
            vals = [plsc.load_gather(table_v, [pos + hh])
                    for hh in range(h)]
            for hh in range(h):
                plsc.store_scatter(out_v, [obase + hh], vals[hh])

        co = pltpu.async_copy(out_v, o_hbm.at[wid], sem_i)
        co.wait()

    return gather_kernel(table_flat, idx2d)


_BM = 128
_EXP_BLOCK = 640


def _mm_body(n_exp, window, tok_ref, w_ref, b_ref, e_ref, out_ref, wexp_ref):
    @pl.when(pl.program_id(0) == 0)
    def _():
        kb = _EXP_BLOCK // window
        for j in range(n_exp):
            w_slice = w_ref[pl.ds(j * kb, kb), :].astype(jnp.bfloat16)
            wexp_ref[pl.ds(j * _EXP_BLOCK, _EXP_BLOCK), :] = jnp.dot(
                e_ref[...], w_slice, preferred_element_type=jnp.float32
            )

    acc = jnp.dot(
        tok_ref[...],
        wexp_ref[...],
        precision=lax.Precision.DEFAULT,
        preferred_element_type=jnp.float32,
    )
    out_ref[...] = acc + b_ref[...]


def _tc_matmul(tokens, w_reduce, bias_row, window):
    bm, tok_len = tokens.shape
    kw, n_out = w_reduce.shape
    n_exp = tok_len // _EXP_BLOCK
    e_mat = jnp.asarray(
        np.kron(np.eye(_EXP_BLOCK // window, dtype=np.float32),
                np.full((window, 1), 1.0 / window, dtype=np.float32)),
        dtype=jnp.bfloat16)
    return pl.pallas_call(
        functools.partial(_mm_body, n_exp, window),
        grid=(bm // _BM,),
        in_specs=[
            pl.BlockSpec((_BM, tok_len), lambda i: (i, 0)),
            pl.BlockSpec((kw, n_out), lambda i: (0, 0)),
            pl.BlockSpec((1, n_out), lambda i: (0, 0)),
            pl.BlockSpec(e_mat.shape, lambda i: (0, 0)),
        ],
        out_specs=pl.BlockSpec((_BM, n_out), lambda i: (i, 0)),
        out_shape=jax.ShapeDtypeStruct((bm, n_out), jnp.float32),
        scratch_shapes=[pltpu.VMEM((tok_len, n_out), jnp.float32)],
        compiler_params=pltpu.CompilerParams(
            dimension_semantics=("arbitrary",),
        ),
    )(tokens, w_reduce, bias_row, e_mat)


def _add_body(a_ref, b_ref, out_ref):
    out_ref[...] = a_ref[...] + b_ref[...]


def _tc_add(a, b):
    return pl.pallas_call(
        _add_body,
        out_shape=jax.ShapeDtypeStruct(a.shape, jnp.float32),
    )(a, b)


def kernel(semantic_tokens, semantic_pos, pos_table, W_reduce, b_reduce):
    b, tok_len = semantic_tokens.shape
    l = semantic_pos.shape[1]
    h = pos_table.shape[1]
    window = tok_len // W_reduce.shape[0]

    idx = semantic_pos.reshape(-1).astype(jnp.int32)
    pos_emb = _sc_gather(pos_table, idx).reshape(b, l * h)
    mm = _tc_matmul(semantic_tokens, W_reduce, b_reduce.reshape(1, -1),
                    window)
    return _tc_add(mm, pos_emb)
